# baseline (speedup 1.0000x reference)
import functools

import jax
import jax.numpy as jnp
from jax import lax
from jax.experimental import pallas as pl
from jax.experimental.pallas import tpu as pltpu
from jax.experimental.pallas import tpu_sc as plsc

NW = 32
LANES = 16
KCH = 128
DPH = 64
DE = DPH + LANES


def _lrelu(t):
    return jnp.where(t > 0, t, 0.2 * t)


def _enc_body(x_ref, w_ref, ats_ref, atd_ref, he_ref, a_ref, b_ref, ps_ref):
    n = x_ref.shape[0]
    nph = he_ref.shape[0]
    h = jnp.dot(x_ref[...], w_ref[...], preferred_element_type=jnp.float32)
    a = jnp.dot(h, ats_ref[...], preferred_element_type=jnp.float32)
    b = jnp.dot(h, atd_ref[...], preferred_element_type=jnp.float32)
    a_ref[...] = a
    b_ref[...] = b
    amax = jnp.max(a)
    sh = _lrelu(amax + b)
    ps_ref[...] = jnp.exp(_lrelu(a + b) - sh)
    marker = jnp.where(
        lax.broadcasted_iota(jnp.int32, (n, LANES), 1) == 0, 1.0, 0.0)
    for ph in range(nph):
        he_ref[ph] = jnp.concatenate(
            [h[:, ph * DPH:(ph + 1) * DPH], marker], axis=1)


def _dense_stage(x, W, att_s, att_d):
    n = x.shape[0]
    nph = W.shape[1] // DPH
    return pl.pallas_call(
        _enc_body,
        out_shape=[
            jax.ShapeDtypeStruct((nph, n, DE), jnp.float32),
            jax.ShapeDtypeStruct((n, 1), jnp.float32),
            jax.ShapeDtypeStruct((n, 1), jnp.float32),
            jax.ShapeDtypeStruct((n, 1), jnp.float32),
        ],
    )(x, W, att_s.reshape(-1, 1), att_d.reshape(-1, 1))


def _norm_body(num_ref, ps_ref, he_ref, out_ref, *, act):
    nph = num_ref.shape[0]
    v = num_ref[...]
    ps = ps_ref[...]
    he = he_ref[...]
    num = jnp.concatenate(
        [v[ph, 0, :, :DPH] + v[ph, 1, :, :DPH] for ph in range(nph)], axis=1)
    h = jnp.concatenate([he[ph, :, :DPH] for ph in range(nph)], axis=1)
    den = v[0, 0, :, DPH:DPH + 1] + v[0, 1, :, DPH:DPH + 1] + ps + 1e-16
    y = (num + ps * h) / den
    if act:
        y = jnp.where(y > 0, y, jnp.exp(y) - 1.0)
    out_ref[...] = y


def _normalize(num, ps, he, act):
    n = ps.shape[0]
    nph = num.shape[0]
    bn = 1000
    return pl.pallas_call(
        functools.partial(_norm_body, act=act),
        grid=(n // bn,),
        in_specs=[
            pl.BlockSpec((nph, 2, bn, DE), lambda i: (0, 0, i, 0)),
            pl.BlockSpec((bn, 1), lambda i: (i, 0)),
            pl.BlockSpec((nph, bn, DE), lambda i: (0, i, 0)),
        ],
        out_specs=pl.BlockSpec((bn, nph * DPH), lambda i: (i, 0)),
        out_shape=jax.ShapeDtypeStruct((n, nph * DPH), jnp.float32),
    )(num, ps, he)


def _edge_pass(n, nph, nchunks, epc):
    n2 = n + LANES
    nt = nchunks // 4
    rpt = (n // LANES) // 8 * 8
    rem_off = rpt * LANES
    rem = n - rem_off
    nz = rpt // 6
    mesh = plsc.VectorSubcoreMesh(core_axis_name="c", subcore_axis_name="s")

    @functools.partial(
        pl.kernel,
        mesh=mesh,
        compiler_params=pltpu.CompilerParams(
            needs_layout_passes=False, use_tc_tiling_on_sc=False),
        out_type=jax.ShapeDtypeStruct((nph, 2, n, DE), jnp.float32),
        scratch_types=[
            pltpu.VMEM((n2,), jnp.float32),
            pltpu.VMEM((n2,), jnp.float32),
            pltpu.VMEM((nchunks, KCH), jnp.int32),
            [pltpu.VMEM((KCH,), jnp.int32)] * 4,
            [pltpu.VMEM((KCH,), jnp.int32)] * 4,
            pltpu.VMEM((KCH,), jnp.float32),
            pltpu.VMEM((4, KCH, DE), jnp.float32),
            pltpu.VMEM_SHARED((n2, DE), jnp.float32),
            [pltpu.SemaphoreType.DMA] * 4,
            [pltpu.SemaphoreType.DMA] * 4,
        ],
    )
    def edge_kernel(pk_hbm, a_hbm, b_hbm, he_hbm, out_hbm,
                    a_v, b_v, pidx, srcu, dstu, p_v, rows, acc, semg, sems):
        cid = lax.axis_index("c")
        sid = lax.axis_index("s")
        w = cid * 16 + sid

        pltpu.sync_copy(pk_hbm.at[w], pidx)
        pltpu.sync_copy(a_hbm, a_v)
        pltpu.sync_copy(b_hbm, b_v)

        def mstep(i, m):
            return jnp.maximum(m, a_v[pl.ds(i * LANES, LANES)])
        m0 = a_v[pl.ds(0, LANES)]
        m = lax.fori_loop(1, n2 // LANES, mstep, m0)
        ii = lax.iota(jnp.int32, LANES)
        for sft in (8, 4, 2, 1):
            p_v[pl.ds(0, LANES)] = m
            m = jnp.maximum(m, plsc.load_gather(p_v, [ii ^ sft]))
        amax = m

        def unpack(j, s):
            for g in range(KCH // LANES):
                sl = pl.ds(g * LANES, LANES)
                wv = pidx[j, sl]
                srcu[s][sl] = wv & 0xFFFF
                dstu[s][sl] = lax.shift_right_logical(wv, 16)

        def compute_p(j, s):
            for g in range(KCH // LANES):
                sl = pl.ds(g * LANES, LANES)
                sv = srcu[s][sl]
                dv = dstu[s][sl]
                av = plsc.load_gather(a_v, [sv])
                bv = plsc.load_gather(b_v, [dv])
                e = _lrelu(av + bv)
                sh = _lrelu(bv + amax)
                p = jnp.exp(e - sh)
                if g * LANES + LANES > epc:
                    p = jnp.where(
                        lax.iota(jnp.int32, LANES) < epc - g * LANES, p, 0.0)
                p_v[sl] = p

        def scale_rows(s):
            def body(g, carry2):
                for i in range(LANES):
                    kk = g * LANES + i
                    pb = plsc.load_gather(
                        p_v, [jnp.full((LANES,), kk, jnp.int32)])
                    for c in range(DE // LANES):
                        sl = pl.ds(c * LANES, LANES)
                        rows[s, kk, sl] = rows[s, kk, sl] * pb
                return carry2
            lax.fori_loop(0, KCH // LANES, body, 0)

        def launch_gather(tbl, s):
            pltpu.async_copy(tbl.at[srcu[s]], rows.at[s], semg[s])

        def wait_gather(tbl, s):
            pltpu.make_async_copy(
                tbl.at[srcu[s]], rows.at[s], semg[s]).wait()

        def launch_scatter(s):
            pltpu.async_copy(rows.at[s], acc.at[dstu[s]], sems[s],
                             add=True)

        def wait_scatter(s):
            pltpu.make_async_copy(
                rows.at[s], acc.at[dstu[s]], sems[s]).wait()

        for ph in range(nph):
            tbl = he_hbm.at[ph]

            def zrow(r, carry):
                for c in range(DE // LANES):
                    rows[3, r, pl.ds(c * LANES, LANES)] = jnp.zeros(
                        (LANES,), jnp.float32)
                return carry
            lax.fori_loop(0, KCH, zrow, 0)

            def zacc(j, carry):
                off = pl.multiple_of(sid * rpt + j * nz, 8)
                pltpu.sync_copy(rows.at[3].at[pl.ds(0, nz)],
                                acc.at[pl.ds(off, nz)])
                return carry
            lax.fori_loop(0, 6, zacc, 0)

            @pl.when(sid == LANES - 1)
            def _():
                pltpu.sync_copy(rows.at[3].at[pl.ds(0, rem)],
                                acc.at[pl.ds(rem_off, rem)])

            plsc.subcore_barrier()

            unpack(0, 0)
            unpack(1, 1)
            launch_gather(tbl, 0)
            launch_gather(tbl, 1)

            def step(t, carry):
                for s in range(4):
                    j = 4 * t + s
                    s2 = (s + 2) % 4
                    if s < 2:
                        @pl.when(t > 0)
                        def _():
                            wait_scatter(s2)
                        unpack(j + 2, s2)
                        launch_gather(tbl, s2)
                    else:
                        wait_scatter(s2)

                        @pl.when(t < nt - 1)
                        def _():
                            unpack(j + 2, s2)
                            launch_gather(tbl, s2)
                    compute_p(j, s)
                    wait_gather(tbl, s)
                    scale_rows(s)
                    launch_scatter(s)
                return carry

            lax.fori_loop(0, nt, step, 0)
            wait_scatter(2)
            wait_scatter(3)
            plsc.subcore_barrier()

            woff = pl.multiple_of(sid * rpt, 8)
            pltpu.sync_copy(acc.at[pl.ds(woff, rpt)],
                            out_hbm.at[ph].at[cid].at[pl.ds(woff, rpt)])

            @pl.when(sid == LANES - 1)
            def _():
                pltpu.sync_copy(acc.at[pl.ds(rem_off, rem)],
                                out_hbm.at[ph].at[cid].at[pl.ds(rem_off,
                                                                rem)])
            if ph + 1 < nph:
                plsc.subcore_barrier()

    return edge_kernel


def _pad_neg(v):
    return jnp.pad(v.reshape(-1), (0, LANES), constant_values=-1e30)


def kernel(X, edge_index, W1, att1_src, att1_dst, W2, att2_src, att2_dst):
    n, d_in = X.shape
    d_hid = W1.shape[1]
    e = edge_index.shape[1]

    order = jnp.argsort(edge_index[1])
    ss = edge_index[0][order]
    ds = edge_index[1][order]
    epc = KCH - 3
    total_chunks = e // epc
    nchunks = total_chunks // NW
    s2 = ss.reshape(epc, total_chunks).T
    d2 = ds.reshape(epc, total_chunks).T
    c = jnp.arange(total_chunks, dtype=jnp.int32)[:, None]
    k3 = jnp.arange(KCH - epc, dtype=jnp.int32)[None, :]
    padd = n + ((c % 5) * 3 + k3) % LANES
    pads = (c * 7 + k3) % n
    s2p = jnp.concatenate([s2, pads], axis=1)
    d2p = jnp.concatenate([d2, padd], axis=1)
    packed = (s2p | (d2p << 16)).reshape(NW, nchunks, KCH)

    he1, a1, b1, ps1 = _dense_stage(X, W1, att1_src, att1_dst)
    num1 = _edge_pass(n, d_hid // DPH, nchunks, epc)(
        packed, _pad_neg(a1), _pad_neg(b1), he1)
    H = _normalize(num1, ps1, he1, act=False)

    he2, a2, b2, ps2 = _dense_stage(H, W2, att2_src, att2_dst)
    num2 = _edge_pass(n, d_in // DPH, nchunks, epc)(
        packed, _pad_neg(a2), _pad_neg(b2), he2)
    X_rec = _normalize(num2, ps2, he2, act=True)

    return (H, X_rec)


# speedup vs baseline: 1.0499x; 1.0499x over previous
import functools

import jax
import jax.numpy as jnp
from jax import lax
from jax.experimental import pallas as pl
from jax.experimental.pallas import tpu as pltpu
from jax.experimental.pallas import tpu_sc as plsc

NW = 32
LANES = 16
KCH = 128
DPH = 64
DE = DPH + LANES


def _lrelu(t):
    return jnp.where(t > 0, t, 0.2 * t)


def _enc_body(x_ref, w_ref, ats_ref, atd_ref, he_ref, a_ref, b_ref, ps_ref):
    n = x_ref.shape[0]
    nph = he_ref.shape[0]
    h = jnp.dot(x_ref[...], w_ref[...], preferred_element_type=jnp.float32)
    a = jnp.dot(h, ats_ref[...], preferred_element_type=jnp.float32)
    b = jnp.dot(h, atd_ref[...], preferred_element_type=jnp.float32)
    a_ref[...] = a
    b_ref[...] = b
    amax = jnp.max(a)
    sh = _lrelu(amax + b)
    ps_ref[...] = jnp.exp(_lrelu(a + b) - sh)
    marker = jnp.where(
        lax.broadcasted_iota(jnp.int32, (n, LANES), 1) == 0, 1.0, 0.0)
    for ph in range(nph):
        he_ref[ph] = jnp.concatenate(
            [h[:, ph * DPH:(ph + 1) * DPH], marker], axis=1)


def _dense_stage(x, W, att_s, att_d):
    n = x.shape[0]
    nph = W.shape[1] // DPH
    return pl.pallas_call(
        _enc_body,
        out_shape=[
            jax.ShapeDtypeStruct((nph, n, DE), jnp.float32),
            jax.ShapeDtypeStruct((n, 1), jnp.float32),
            jax.ShapeDtypeStruct((n, 1), jnp.float32),
            jax.ShapeDtypeStruct((n, 1), jnp.float32),
        ],
    )(x, W, att_s.reshape(-1, 1), att_d.reshape(-1, 1))


def _norm_body(num_ref, ps_ref, he_ref, out_ref, *, act):
    nph = num_ref.shape[0]
    v = num_ref[...]
    ps = ps_ref[...]
    he = he_ref[...]
    num = jnp.concatenate(
        [v[ph, 0, :, :DPH] + v[ph, 1, :, :DPH] for ph in range(nph)], axis=1)
    h = jnp.concatenate([he[ph, :, :DPH] for ph in range(nph)], axis=1)
    den = v[0, 0, :, DPH:DPH + 1] + v[0, 1, :, DPH:DPH + 1] + ps + 1e-16
    y = (num + ps * h) / den
    if act:
        y = jnp.where(y > 0, y, jnp.exp(y) - 1.0)
    out_ref[...] = y


def _normalize(num, ps, he, act):
    n = ps.shape[0]
    nph = num.shape[0]
    bn = 1000
    return pl.pallas_call(
        functools.partial(_norm_body, act=act),
        grid=(n // bn,),
        in_specs=[
            pl.BlockSpec((nph, 2, bn, DE), lambda i: (0, 0, i, 0)),
            pl.BlockSpec((bn, 1), lambda i: (i, 0)),
            pl.BlockSpec((nph, bn, DE), lambda i: (0, i, 0)),
        ],
        out_specs=pl.BlockSpec((bn, nph * DPH), lambda i: (i, 0)),
        out_shape=jax.ShapeDtypeStruct((n, nph * DPH), jnp.float32),
    )(num, ps, he)


def _edge_pass(n, nph, nchunks, epc):
    n2 = n + LANES
    nt = nchunks // 4
    rpt = (n // LANES) // 8 * 8
    rem_off = rpt * LANES
    rem = n - rem_off
    nz = rpt // 6
    mesh = plsc.VectorSubcoreMesh(core_axis_name="c", subcore_axis_name="s")

    @functools.partial(
        pl.kernel,
        mesh=mesh,
        compiler_params=pltpu.CompilerParams(
            needs_layout_passes=False, use_tc_tiling_on_sc=False),
        out_type=jax.ShapeDtypeStruct((nph, 2, n, DE), jnp.float32),
        scratch_types=[
            pltpu.VMEM((n2,), jnp.float32),
            pltpu.VMEM((n2,), jnp.float32),
            pltpu.VMEM((nchunks, KCH), jnp.int32),
            [pltpu.VMEM((KCH,), jnp.int32)] * 4,
            [pltpu.VMEM((KCH,), jnp.int32)] * 4,
            pltpu.VMEM((KCH,), jnp.float32),
            pltpu.VMEM((4, KCH, DE), jnp.float32),
            pltpu.VMEM_SHARED((n2, DE), jnp.float32),
            [pltpu.SemaphoreType.DMA] * 4,
            [pltpu.SemaphoreType.DMA] * 4,
        ],
    )
    def edge_kernel(pk_hbm, a_hbm, b_hbm, he_hbm, out_hbm,
                    a_v, b_v, pidx, srcu, dstu, p_v, rows, acc, semg, sems):
        cid = lax.axis_index("c")
        sid = lax.axis_index("s")
        w = cid * 16 + sid

        pltpu.sync_copy(pk_hbm.at[w], pidx)
        pltpu.sync_copy(a_hbm, a_v)
        pltpu.sync_copy(b_hbm, b_v)

        def mstep(i, m):
            return jnp.maximum(m, a_v[pl.ds(i * LANES, LANES)])
        m0 = a_v[pl.ds(0, LANES)]
        m = lax.fori_loop(1, n2 // LANES, mstep, m0)
        ii = lax.iota(jnp.int32, LANES)
        for sft in (8, 4, 2, 1):
            p_v[pl.ds(0, LANES)] = m
            m = jnp.maximum(m, plsc.load_gather(p_v, [ii ^ sft]))
        amax = m

        def unpack(j, s):
            for g in range(KCH // LANES):
                sl = pl.ds(g * LANES, LANES)
                wv = pidx[j, sl]
                srcu[s][sl] = wv & 0x3FFF
                dstu[s][sl] = lax.shift_right_logical(wv, 14)

        def compute_p(j, s):
            for g in range(KCH // LANES):
                sl = pl.ds(g * LANES, LANES)
                sv = srcu[s][sl]
                dv = dstu[s][sl]
                av = plsc.load_gather(a_v, [sv])
                bv = plsc.load_gather(b_v, [dv])
                e = _lrelu(av + bv)
                sh = _lrelu(bv + amax)
                p = jnp.exp(e - sh)
                if g * LANES + LANES > epc:
                    p = jnp.where(
                        lax.iota(jnp.int32, LANES) < epc - g * LANES, p, 0.0)
                p_v[sl] = p

        def scale_rows(s):
            def body(g, carry2):
                for i in range(LANES):
                    kk = g * LANES + i
                    pb = plsc.load_gather(
                        p_v, [jnp.full((LANES,), kk, jnp.int32)])
                    for c in range(DE // LANES):
                        sl = pl.ds(c * LANES, LANES)
                        rows[s, kk, sl] = rows[s, kk, sl] * pb
                return carry2
            lax.fori_loop(0, KCH // LANES, body, 0)

        def launch_gather(tbl, s):
            pltpu.async_copy(tbl.at[srcu[s]], rows.at[s], semg[s])

        def wait_gather(tbl, s):
            pltpu.make_async_copy(
                tbl.at[srcu[s]], rows.at[s], semg[s]).wait()

        def launch_scatter(s):
            pltpu.async_copy(rows.at[s], acc.at[dstu[s]], sems[s],
                             add=True)

        def wait_scatter(s):
            pltpu.make_async_copy(
                rows.at[s], acc.at[dstu[s]], sems[s]).wait()

        for ph in range(nph):
            tbl = he_hbm.at[ph]

            def zrow(r, carry):
                for c in range(DE // LANES):
                    rows[3, r, pl.ds(c * LANES, LANES)] = jnp.zeros(
                        (LANES,), jnp.float32)
                return carry
            lax.fori_loop(0, KCH, zrow, 0)

            def zacc(j, carry):
                off = pl.multiple_of(sid * rpt + j * nz, 8)
                pltpu.sync_copy(rows.at[3].at[pl.ds(0, nz)],
                                acc.at[pl.ds(off, nz)])
                return carry
            lax.fori_loop(0, 6, zacc, 0)

            @pl.when(sid == LANES - 1)
            def _():
                pltpu.sync_copy(rows.at[3].at[pl.ds(0, rem)],
                                acc.at[pl.ds(rem_off, rem)])

            plsc.subcore_barrier()

            unpack(0, 0)
            unpack(1, 1)
            launch_gather(tbl, 0)
            launch_gather(tbl, 1)

            def step(t, carry):
                for s in range(4):
                    j = 4 * t + s
                    s2 = (s + 2) % 4
                    if s < 2:
                        @pl.when(t > 0)
                        def _():
                            wait_scatter(s2)
                        unpack(j + 2, s2)
                        launch_gather(tbl, s2)
                    else:
                        wait_scatter(s2)

                        @pl.when(t < nt - 1)
                        def _():
                            unpack(j + 2, s2)
                            launch_gather(tbl, s2)
                    compute_p(j, s)
                    wait_gather(tbl, s)
                    scale_rows(s)
                    launch_scatter(s)
                return carry

            lax.fori_loop(0, nt, step, 0)
            wait_scatter(2)
            wait_scatter(3)
            plsc.subcore_barrier()

            woff = pl.multiple_of(sid * rpt, 8)
            pltpu.sync_copy(acc.at[pl.ds(woff, rpt)],
                            out_hbm.at[ph].at[cid].at[pl.ds(woff, rpt)])

            @pl.when(sid == LANES - 1)
            def _():
                pltpu.sync_copy(acc.at[pl.ds(rem_off, rem)],
                                out_hbm.at[ph].at[cid].at[pl.ds(rem_off,
                                                                rem)])
            if ph + 1 < nph:
                plsc.subcore_barrier()

    return edge_kernel


def _pad_neg(v):
    return jnp.pad(v.reshape(-1), (0, LANES), constant_values=-1e30)


def kernel(X, edge_index, W1, att1_src, att1_dst, W2, att2_src, att2_dst):
    n, d_in = X.shape
    d_hid = W1.shape[1]
    e = edge_index.shape[1]

    key = (edge_index[1] << 14) | edge_index[0]
    ks = jnp.sort(key)
    epc = KCH - 3
    total_chunks = e // epc
    nchunks = total_chunks // NW
    st = ks.reshape(epc, total_chunks).T
    c = jnp.arange(total_chunks, dtype=jnp.int32)[:, None]
    k3 = jnp.arange(KCH - epc, dtype=jnp.int32)[None, :]
    padd = n + ((c % 5) * 3 + k3) % LANES
    pads = (c * 7 + k3) % n
    padv = (padd << 14) | pads
    packed = jnp.concatenate([st, padv], axis=1).reshape(NW, nchunks, KCH)

    he1, a1, b1, ps1 = _dense_stage(X, W1, att1_src, att1_dst)
    num1 = _edge_pass(n, d_hid // DPH, nchunks, epc)(
        packed, _pad_neg(a1), _pad_neg(b1), he1)
    H = _normalize(num1, ps1, he1, act=False)

    he2, a2, b2, ps2 = _dense_stage(H, W2, att2_src, att2_dst)
    num2 = _edge_pass(n, d_in // DPH, nchunks, epc)(
        packed, _pad_neg(a2), _pad_neg(b2), he2)
    X_rec = _normalize(num2, ps2, he2, act=True)

    return (H, X_rec)


# speedup vs baseline: 1.0804x; 1.0291x over previous
import functools

import jax
import jax.numpy as jnp
from jax import lax
from jax.experimental import pallas as pl
from jax.experimental.pallas import tpu as pltpu
from jax.experimental.pallas import tpu_sc as plsc

NW = 32
LANES = 16
KCH = 128
DPH = 64
DE = DPH + LANES


def _lrelu(t):
    return jnp.where(t > 0, t, 0.2 * t)


def _enc_body(x_ref, w_ref, ats_ref, atd_ref, he_ref, a_ref, b_ref, ps_ref):
    n = x_ref.shape[0]
    nph = he_ref.shape[0]
    h = jnp.dot(x_ref[...], w_ref[...], preferred_element_type=jnp.float32)
    a = jnp.dot(h, ats_ref[...], preferred_element_type=jnp.float32)
    b = jnp.dot(h, atd_ref[...], preferred_element_type=jnp.float32)
    a_ref[...] = a
    b_ref[...] = b
    amax = jnp.max(a)
    sh = _lrelu(amax + b)
    ps_ref[...] = jnp.exp(_lrelu(a + b) - sh)
    marker = jnp.where(
        lax.broadcasted_iota(jnp.int32, (n, LANES), 1) == 0, 1.0, 0.0)
    for ph in range(nph):
        he_ref[ph] = jnp.concatenate(
            [h[:, ph * DPH:(ph + 1) * DPH], marker], axis=1)


def _dense_stage(x, W, att_s, att_d):
    n = x.shape[0]
    nph = W.shape[1] // DPH
    return pl.pallas_call(
        _enc_body,
        out_shape=[
            jax.ShapeDtypeStruct((nph, n, DE), jnp.float32),
            jax.ShapeDtypeStruct((n, 1), jnp.float32),
            jax.ShapeDtypeStruct((n, 1), jnp.float32),
            jax.ShapeDtypeStruct((n, 1), jnp.float32),
        ],
    )(x, W, att_s.reshape(-1, 1), att_d.reshape(-1, 1))


def _norm_body(num_ref, ps_ref, he_ref, out_ref, *, act):
    nph = num_ref.shape[0]
    v = num_ref[...]
    ps = ps_ref[...]
    he = he_ref[...]
    num = jnp.concatenate(
        [v[ph, 0, :, :DPH] + v[ph, 1, :, :DPH] for ph in range(nph)], axis=1)
    h = jnp.concatenate([he[ph, :, :DPH] for ph in range(nph)], axis=1)
    den = v[0, 0, :, DPH:DPH + 1] + v[0, 1, :, DPH:DPH + 1] + ps + 1e-16
    y = (num + ps * h) / den
    if act:
        y = jnp.where(y > 0, y, jnp.exp(y) - 1.0)
    out_ref[...] = y


def _normalize(num, ps, he, act):
    n = ps.shape[0]
    nph = num.shape[0]
    bn = 1000
    return pl.pallas_call(
        functools.partial(_norm_body, act=act),
        grid=(n // bn,),
        in_specs=[
            pl.BlockSpec((nph, 2, bn, DE), lambda i: (0, 0, i, 0)),
            pl.BlockSpec((bn, 1), lambda i: (i, 0)),
            pl.BlockSpec((nph, bn, DE), lambda i: (0, i, 0)),
        ],
        out_specs=pl.BlockSpec((bn, nph * DPH), lambda i: (i, 0)),
        out_shape=jax.ShapeDtypeStruct((n, nph * DPH), jnp.float32),
    )(num, ps, he)


def _edge_pass(n, nph, nchunks, epc):
    n2 = n + LANES
    nt = nchunks // 4
    rpt = (n // LANES) // 8 * 8
    rem_off = rpt * LANES
    rem = n - rem_off
    nz = rpt // 6
    mesh = plsc.VectorSubcoreMesh(core_axis_name="c", subcore_axis_name="s")

    @functools.partial(
        pl.kernel,
        mesh=mesh,
        compiler_params=pltpu.CompilerParams(
            needs_layout_passes=False, use_tc_tiling_on_sc=False),
        out_type=jax.ShapeDtypeStruct((nph, 2, n, DE), jnp.float32),
        scratch_types=[
            pltpu.VMEM((n2,), jnp.float32),
            pltpu.VMEM((n2,), jnp.float32),
            pltpu.VMEM((nchunks, KCH), jnp.int32),
            [pltpu.VMEM((KCH,), jnp.int32)] * 4,
            [pltpu.VMEM((KCH,), jnp.int32)] * 4,
            pltpu.VMEM((KCH,), jnp.float32),
            pltpu.VMEM((4, KCH, DE), jnp.float32),
            pltpu.VMEM_SHARED((n2, DE), jnp.float32),
            [pltpu.SemaphoreType.DMA] * 4,
            [pltpu.SemaphoreType.DMA] * 4,
        ],
    )
    def edge_kernel(pk_hbm, a_hbm, b_hbm, he_hbm, out_hbm,
                    a_v, b_v, pidx, srcu, dstu, p_v, rows, acc, semg, sems):
        cid = lax.axis_index("c")
        sid = lax.axis_index("s")
        w = cid * 16 + sid

        pltpu.sync_copy(pk_hbm.at[w], pidx)
        pltpu.sync_copy(a_hbm, a_v)
        pltpu.sync_copy(b_hbm, b_v)

        def mstep(i, m):
            return jnp.maximum(m, a_v[pl.ds(i * LANES, LANES)])
        m0 = a_v[pl.ds(0, LANES)]
        m = lax.fori_loop(1, n2 // LANES, mstep, m0)
        ii = lax.iota(jnp.int32, LANES)
        for sft in (8, 4, 2, 1):
            p_v[pl.ds(0, LANES)] = m
            m = jnp.maximum(m, plsc.load_gather(p_v, [ii ^ sft]))
        amax = m

        def unpack(j, s):
            for g in range(KCH // LANES):
                sl = pl.ds(g * LANES, LANES)
                wv = pidx[j, sl]
                srcu[s][sl] = wv & 0x3FFF
                dstu[s][sl] = lax.shift_right_logical(wv, 14)

        def compute_p(j, s):
            for g in range(KCH // LANES):
                sl = pl.ds(g * LANES, LANES)
                sv = srcu[s][sl]
                dv = dstu[s][sl]
                av = plsc.load_gather(a_v, [sv])
                bv = plsc.load_gather(b_v, [dv])
                e = _lrelu(av + bv)
                sh = _lrelu(bv + amax)
                p = jnp.exp(e - sh)
                if g * LANES + LANES > epc:
                    p = jnp.where(
                        lax.iota(jnp.int32, LANES) < epc - g * LANES, p, 0.0)
                p_v[sl] = p

        def scale_rows(s):
            def body(g, carry2):
                for i in range(LANES):
                    kk = g * LANES + i
                    pb = plsc.load_gather(
                        p_v, [jnp.full((LANES,), kk, jnp.int32)])
                    for c in range(DE // LANES):
                        sl = pl.ds(c * LANES, LANES)
                        rows[s, kk, sl] = rows[s, kk, sl] * pb
                return carry2
            lax.fori_loop(0, KCH // LANES, body, 0)

        def launch_gather(tbl, s):
            pltpu.async_copy(tbl.at[srcu[s]], rows.at[s], semg[s])

        def wait_gather(tbl, s):
            pltpu.make_async_copy(
                tbl.at[srcu[s]], rows.at[s], semg[s]).wait()

        def launch_scatter(s):
            pltpu.async_copy(rows.at[s], acc.at[dstu[s]], sems[s],
                             add=True)

        def wait_scatter(s):
            pltpu.make_async_copy(
                rows.at[s], acc.at[dstu[s]], sems[s]).wait()

        for ph in range(nph):
            tbl = he_hbm.at[ph]

            def zrow(r, carry):
                for c in range(DE // LANES):
                    rows[3, r, pl.ds(c * LANES, LANES)] = jnp.zeros(
                        (LANES,), jnp.float32)
                return carry
            lax.fori_loop(0, KCH, zrow, 0)

            def zacc(j, carry):
                off = pl.multiple_of(sid * rpt + j * nz, 8)
                pltpu.sync_copy(rows.at[3].at[pl.ds(0, nz)],
                                acc.at[pl.ds(off, nz)])
                return carry
            lax.fori_loop(0, 6, zacc, 0)

            @pl.when(sid == LANES - 1)
            def _():
                pltpu.sync_copy(rows.at[3].at[pl.ds(0, rem)],
                                acc.at[pl.ds(rem_off, rem)])

            plsc.subcore_barrier()

            unpack(0, 0)
            unpack(1, 1)
            launch_gather(tbl, 0)
            launch_gather(tbl, 1)

            def step(t, carry):
                for s in range(4):
                    j = 4 * t + s
                    s2 = (s + 2) % 4
                    if s < 2:
                        @pl.when(t > 0)
                        def _():
                            wait_scatter(s2)
                        unpack(j + 2, s2)
                        launch_gather(tbl, s2)
                    else:
                        wait_scatter(s2)

                        @pl.when(t < nt - 1)
                        def _():
                            unpack(j + 2, s2)
                            launch_gather(tbl, s2)
                    compute_p(j, s)
                    wait_gather(tbl, s)
                    scale_rows(s)
                    launch_scatter(s)
                return carry

            lax.fori_loop(0, nt, step, 0)
            wait_scatter(2)
            wait_scatter(3)
            plsc.subcore_barrier()

            woff = pl.multiple_of(sid * rpt, 8)
            pltpu.sync_copy(acc.at[pl.ds(woff, rpt)],
                            out_hbm.at[ph].at[cid].at[pl.ds(woff, rpt)])

            @pl.when(sid == LANES - 1)
            def _():
                pltpu.sync_copy(acc.at[pl.ds(rem_off, rem)],
                                out_hbm.at[ph].at[cid].at[pl.ds(rem_off,
                                                                rem)])
            if ph + 1 < nph:
                plsc.subcore_barrier()

    return edge_kernel


def _pad_neg(v):
    return jnp.pad(v.reshape(-1), (0, LANES), constant_values=-1e30)


def kernel(X, edge_index, W1, att1_src, att1_dst, W2, att2_src, att2_dst):
    n, d_in = X.shape
    d_hid = W1.shape[1]
    e = edge_index.shape[1]

    key = (edge_index[1] << 14) | edge_index[0]
    epc = KCH - 3
    total_chunks = e // epc
    nchunks = total_chunks // NW
    ks = jnp.sort(key.reshape(NW, e // NW), axis=1)
    st = jnp.swapaxes(ks.reshape(NW, epc, nchunks), 1, 2)
    c = jnp.arange(total_chunks, dtype=jnp.int32)[:, None]
    k3 = jnp.arange(KCH - epc, dtype=jnp.int32)[None, :]
    padd = n + ((c % 5) * 3 + k3) % LANES
    pads = (c * 7 + k3) % n
    padv = ((padd << 14) | pads).reshape(NW, nchunks, KCH - epc)
    packed = jnp.concatenate([st, padv], axis=2)

    he1, a1, b1, ps1 = _dense_stage(X, W1, att1_src, att1_dst)
    num1 = _edge_pass(n, d_hid // DPH, nchunks, epc)(
        packed, _pad_neg(a1), _pad_neg(b1), he1)
    H = _normalize(num1, ps1, he1, act=False)

    he2, a2, b2, ps2 = _dense_stage(H, W2, att2_src, att2_dst)
    num2 = _edge_pass(n, d_in // DPH, nchunks, epc)(
        packed, _pad_neg(a2), _pad_neg(b2), he2)
    X_rec = _normalize(num2, ps2, he2, act=True)

    return (H, X_rec)


# speedup vs baseline: 1.3774x; 1.2749x over previous
import functools

import jax
import jax.numpy as jnp
from jax import lax
from jax.experimental import pallas as pl
from jax.experimental.pallas import tpu as pltpu
from jax.experimental.pallas import tpu_sc as plsc

NW = 32
LANES = 16
KCH = 128
DPH = 64
DE = DPH + LANES


def _lrelu(t):
    return jnp.where(t > 0, t, 0.2 * t)


def _enc_body(x_ref, w_ref, ats_ref, atd_ref, he_ref, a_ref, b_ref, ps_ref):
    n = x_ref.shape[0]
    nph = he_ref.shape[0]
    h = jnp.dot(x_ref[...], w_ref[...], preferred_element_type=jnp.float32)
    a = jnp.dot(h, ats_ref[...], preferred_element_type=jnp.float32)
    b = jnp.dot(h, atd_ref[...], preferred_element_type=jnp.float32)
    a_ref[...] = a
    b_ref[...] = b
    amax = jnp.max(a)
    sh = _lrelu(amax + b)
    ps_ref[...] = jnp.exp(_lrelu(a + b) - sh)
    marker = jnp.where(
        lax.broadcasted_iota(jnp.int32, (n, LANES), 1) == 0, 1.0, 0.0)
    for ph in range(nph):
        he_ref[ph] = jnp.concatenate(
            [h[:, ph * DPH:(ph + 1) * DPH], marker], axis=1)


def _dense_stage(x, W, att_s, att_d):
    n = x.shape[0]
    nph = W.shape[1] // DPH
    return pl.pallas_call(
        _enc_body,
        out_shape=[
            jax.ShapeDtypeStruct((nph, n, DE), jnp.float32),
            jax.ShapeDtypeStruct((n, 1), jnp.float32),
            jax.ShapeDtypeStruct((n, 1), jnp.float32),
            jax.ShapeDtypeStruct((n, 1), jnp.float32),
        ],
    )(x, W, att_s.reshape(-1, 1), att_d.reshape(-1, 1))


def _norm_body(num_ref, ps_ref, he_ref, out_ref, *, act):
    nph = num_ref.shape[0]
    v = num_ref[...]
    ps = ps_ref[...]
    he = he_ref[...]
    num = jnp.concatenate(
        [v[ph, 0, :, :DPH] + v[ph, 1, :, :DPH] for ph in range(nph)], axis=1)
    h = jnp.concatenate([he[ph, :, :DPH] for ph in range(nph)], axis=1)
    den = v[0, 0, :, DPH:DPH + 1] + v[0, 1, :, DPH:DPH + 1] + ps + 1e-16
    y = (num + ps * h) / den
    if act:
        y = jnp.where(y > 0, y, jnp.exp(y) - 1.0)
    out_ref[...] = y


def _normalize(num, ps, he, act):
    n = ps.shape[0]
    nph = num.shape[0]
    bn = 1000
    return pl.pallas_call(
        functools.partial(_norm_body, act=act),
        grid=(n // bn,),
        in_specs=[
            pl.BlockSpec((nph, 2, bn, DE), lambda i: (0, 0, i, 0)),
            pl.BlockSpec((bn, 1), lambda i: (i, 0)),
            pl.BlockSpec((nph, bn, DE), lambda i: (0, i, 0)),
        ],
        out_specs=pl.BlockSpec((bn, nph * DPH), lambda i: (i, 0)),
        out_shape=jax.ShapeDtypeStruct((n, nph * DPH), jnp.float32),
    )(num, ps, he)


def _edge_pass(n, nph, nchunks, epc):
    n2 = n + LANES
    nt = nchunks // 4
    rpt = (n // LANES) // 8 * 8
    rem_off = rpt * LANES
    rem = n - rem_off
    nz = rpt // 6
    mesh = plsc.VectorSubcoreMesh(core_axis_name="c", subcore_axis_name="s")

    @functools.partial(
        pl.kernel,
        mesh=mesh,
        compiler_params=pltpu.CompilerParams(
            needs_layout_passes=False, use_tc_tiling_on_sc=False),
        out_type=jax.ShapeDtypeStruct((nph, 2, n, DE), jnp.float32),
        scratch_types=[
            pltpu.VMEM((n2,), jnp.float32),
            pltpu.VMEM((n2,), jnp.float32),
            pltpu.VMEM((nchunks, KCH), jnp.int32),
            [pltpu.VMEM((KCH,), jnp.int32)] * 4,
            [pltpu.VMEM((KCH,), jnp.int32)] * 4,
            pltpu.VMEM((KCH,), jnp.float32),
            pltpu.VMEM((4, KCH, DE), jnp.float32),
            pltpu.VMEM_SHARED((n2, DE), jnp.float32),
            [pltpu.SemaphoreType.DMA] * 4,
        ],
    )
    def edge_kernel(pk_hbm, a_hbm, b_hbm, he_hbm, out_hbm,
                    a_v, b_v, pidx, srcu, dstu, p_v, rows, acc, semg):
        cid = lax.axis_index("c")
        sid = lax.axis_index("s")
        w = cid * 16 + sid

        pltpu.sync_copy(pk_hbm.at[w], pidx)
        pltpu.sync_copy(a_hbm, a_v)
        pltpu.sync_copy(b_hbm, b_v)

        def mstep(i, m):
            return jnp.maximum(m, a_v[pl.ds(i * LANES, LANES)])
        m0 = a_v[pl.ds(0, LANES)]
        m = lax.fori_loop(1, n2 // LANES, mstep, m0)
        ii = lax.iota(jnp.int32, LANES)
        for sft in (8, 4, 2, 1):
            p_v[pl.ds(0, LANES)] = m
            m = jnp.maximum(m, plsc.load_gather(p_v, [ii ^ sft]))
        amax = m

        def unpack(j, s):
            for g in range(KCH // LANES):
                sl = pl.ds(g * LANES, LANES)
                wv = pidx[j, sl]
                srcu[s][sl] = wv & 0x3FFF
                dstu[s][sl] = lax.shift_right_logical(wv, 14)

        def compute_p(j, s):
            for g in range(KCH // LANES):
                sl = pl.ds(g * LANES, LANES)
                sv = srcu[s][sl]
                dv = dstu[s][sl]
                av = plsc.load_gather(a_v, [sv])
                bv = plsc.load_gather(b_v, [dv])
                e = _lrelu(av + bv)
                sh = _lrelu(bv + amax)
                p = jnp.exp(e - sh)
                if g * LANES + LANES > epc:
                    p = jnp.where(
                        lax.iota(jnp.int32, LANES) < epc - g * LANES, p, 0.0)
                p_v[sl] = p

        def scale_rows(s):
            def body(g, carry2):
                for i in range(LANES):
                    kk = g * LANES + i
                    pb = plsc.load_gather(
                        p_v, [jnp.full((LANES,), kk, jnp.int32)])
                    for c in range(DE // LANES):
                        sl = pl.ds(c * LANES, LANES)
                        rows[s, kk, sl] = rows[s, kk, sl] * pb
                return carry2
            lax.fori_loop(0, KCH // LANES, body, 0)

        def launch_gather(tbl, s):
            pltpu.async_copy(tbl.at[srcu[s]], rows.at[s], semg[s])

        def wait_gather(tbl, s):
            pltpu.make_async_copy(
                tbl.at[srcu[s]], rows.at[s], semg[s]).wait()

        def launch_scatter(s):
            pltpu.sync_copy(rows.at[s], acc.at[dstu[s]], add=True)

        for ph in range(nph):
            tbl = he_hbm.at[ph]

            def zrow(r, carry):
                for c in range(DE // LANES):
                    rows[3, r, pl.ds(c * LANES, LANES)] = jnp.zeros(
                        (LANES,), jnp.float32)
                return carry
            lax.fori_loop(0, KCH, zrow, 0)

            def zacc(j, carry):
                off = pl.multiple_of(sid * rpt + j * nz, 8)
                pltpu.sync_copy(rows.at[3].at[pl.ds(0, nz)],
                                acc.at[pl.ds(off, nz)])
                return carry
            lax.fori_loop(0, 6, zacc, 0)

            @pl.when(sid == LANES - 1)
            def _():
                pltpu.sync_copy(rows.at[3].at[pl.ds(0, rem)],
                                acc.at[pl.ds(rem_off, rem)])

            plsc.subcore_barrier()

            unpack(0, 0)
            unpack(1, 1)
            launch_gather(tbl, 0)
            launch_gather(tbl, 1)

            def step(t, carry):
                for s in range(4):
                    j = 4 * t + s
                    s2 = (s + 2) % 4
                    if s < 2:
                        unpack(j + 2, s2)
                        launch_gather(tbl, s2)
                    else:
                        @pl.when(t < nt - 1)
                        def _():
                            unpack(j + 2, s2)
                            launch_gather(tbl, s2)
                    compute_p(j, s)
                    wait_gather(tbl, s)
                    scale_rows(s)
                    launch_scatter(s)
                return carry

            lax.fori_loop(0, nt, step, 0)
            plsc.subcore_barrier()

            woff = pl.multiple_of(sid * rpt, 8)
            pltpu.sync_copy(acc.at[pl.ds(woff, rpt)],
                            out_hbm.at[ph].at[cid].at[pl.ds(woff, rpt)])

            @pl.when(sid == LANES - 1)
            def _():
                pltpu.sync_copy(acc.at[pl.ds(rem_off, rem)],
                                out_hbm.at[ph].at[cid].at[pl.ds(rem_off,
                                                                rem)])
            if ph + 1 < nph:
                plsc.subcore_barrier()

    return edge_kernel


def _pad_neg(v):
    return jnp.pad(v.reshape(-1), (0, LANES), constant_values=-1e30)


def kernel(X, edge_index, W1, att1_src, att1_dst, W2, att2_src, att2_dst):
    n, d_in = X.shape
    d_hid = W1.shape[1]
    e = edge_index.shape[1]

    key = (edge_index[1] << 14) | edge_index[0]
    epc = KCH - 3
    total_chunks = e // epc
    nchunks = total_chunks // NW
    ks = key.reshape(NW, e // NW)
    st = jnp.swapaxes(ks.reshape(NW, epc, nchunks), 1, 2)
    c = jnp.arange(total_chunks, dtype=jnp.int32)[:, None]
    k3 = jnp.arange(KCH - epc, dtype=jnp.int32)[None, :]
    padd = n + ((c % 5) * 3 + k3) % LANES
    pads = (c * 7 + k3) % n
    padv = ((padd << 14) | pads).reshape(NW, nchunks, KCH - epc)
    packed = jnp.concatenate([st, padv], axis=2)

    he1, a1, b1, ps1 = _dense_stage(X, W1, att1_src, att1_dst)
    num1 = _edge_pass(n, d_hid // DPH, nchunks, epc)(
        packed, _pad_neg(a1), _pad_neg(b1), he1)
    H = _normalize(num1, ps1, he1, act=False)

    he2, a2, b2, ps2 = _dense_stage(H, W2, att2_src, att2_dst)
    num2 = _edge_pass(n, d_in // DPH, nchunks, epc)(
        packed, _pad_neg(a2), _pad_neg(b2), he2)
    X_rec = _normalize(num2, ps2, he2, act=True)

    return (H, X_rec)
